# baseline (speedup 1.0000x reference)
import functools

import jax
import jax.numpy as jnp
from jax.experimental import pallas as pl
from jax.experimental.pallas import tpu as pltpu

SEQ = 2048
D_MODEL = 1024
EXPERT_DIM = 2048
NUM_EXPERTS = 8
BT = 512


def _dense_body(x_ref, w1_ref, b1_ref, w2_ref, b2_ref, gw_ref, gb_ref,
                out_ref, comb_ref):
    e = pl.program_id(1)

    @pl.when(e == 0)
    def _gating():
        x = x_ref[...]
        logits = jnp.dot(x, gw_ref[...], preferred_element_type=jnp.float32)
        logits = logits + gb_ref[...]
        iota = jax.lax.broadcasted_iota(jnp.int32, (BT, NUM_EXPERTS), 1)
        m1 = jnp.max(logits, axis=-1, keepdims=True)
        is1 = logits == m1
        idx1 = jnp.min(jnp.where(is1, iota, NUM_EXPERTS), axis=-1,
                       keepdims=True)
        neg = jnp.float32(-jnp.inf)
        masked = jnp.where(iota == idx1, neg, logits)
        m2 = jnp.max(masked, axis=-1, keepdims=True)
        is2 = masked == m2
        idx2 = jnp.min(jnp.where(is2, iota, NUM_EXPERTS), axis=-1,
                       keepdims=True)
        e2 = jnp.exp(m2 - m1)
        s = 1.0 + e2
        w1 = 1.0 / s
        w2 = e2 / s
        comb_ref[...] = (jnp.where(iota == idx1, w1, 0.0)
                         + jnp.where(iota == idx2, w2, 0.0))

    x = x_ref[...]
    h = jnp.dot(x.astype(jnp.bfloat16), w1_ref[0],
                preferred_element_type=jnp.float32) + b1_ref[0]
    h = jnp.maximum(h, 0.0)
    o = jnp.dot(h.astype(jnp.bfloat16), w2_ref[0],
                preferred_element_type=jnp.float32) + b2_ref[0]
    comb = comb_ref[...]
    eio = jax.lax.broadcasted_iota(jnp.int32, (BT, NUM_EXPERTS), 1)
    w_e = jnp.sum(jnp.where(eio == e, comb, 0.0), axis=-1, keepdims=True)
    contrib = w_e * o

    @pl.when(e == 0)
    def _init():
        out_ref[...] = contrib

    @pl.when(e != 0)
    def _acc():
        out_ref[...] += contrib


@functools.partial(jax.jit, static_argnums=())
def _dense_moe(x_flat, W1, b1, W2, b2, gate_w, gb):
    T = x_flat.shape[0]
    grid = (T // BT, NUM_EXPERTS)
    return pl.pallas_call(
        _dense_body,
        grid=grid,
        in_specs=[
            pl.BlockSpec((BT, D_MODEL), lambda t, e: (t, 0)),
            pl.BlockSpec((1, D_MODEL, EXPERT_DIM), lambda t, e: (e, 0, 0)),
            pl.BlockSpec((1, 1, EXPERT_DIM), lambda t, e: (e, 0, 0)),
            pl.BlockSpec((1, EXPERT_DIM, D_MODEL), lambda t, e: (e, 0, 0)),
            pl.BlockSpec((1, 1, D_MODEL), lambda t, e: (e, 0, 0)),
            pl.BlockSpec((D_MODEL, NUM_EXPERTS), lambda t, e: (0, 0)),
            pl.BlockSpec((1, NUM_EXPERTS), lambda t, e: (0, 0)),
        ],
        out_specs=pl.BlockSpec((BT, D_MODEL), lambda t, e: (t, 0)),
        out_shape=jax.ShapeDtypeStruct((T, D_MODEL), jnp.float32),
        scratch_shapes=[pltpu.VMEM((BT, NUM_EXPERTS), jnp.float32)],
    )(x_flat, W1.astype(jnp.bfloat16), b1.reshape(NUM_EXPERTS, 1, EXPERT_DIM),
      W2.astype(jnp.bfloat16), b2.reshape(NUM_EXPERTS, 1, D_MODEL), gate_w, gb)


def kernel(x, W1, b1, W2, b2, gate_w, gate_b, bias):
    seq_len, batch_size, d_model = x.shape
    x_flat = x.reshape(-1, d_model)
    gb = (gate_b + bias).reshape(1, NUM_EXPERTS)
    out = _dense_moe(x_flat, W1, b1, W2, b2, gate_w, gb)
    return out.reshape(seq_len, batch_size, d_model)


# speedup vs baseline: 1.1555x; 1.1555x over previous
import functools

import jax
import jax.numpy as jnp
from jax import lax
from jax.experimental import pallas as pl
from jax.experimental.pallas import tpu as pltpu
from jax.experimental.pallas import tpu_sc as plsc

T = 2048
D_MODEL = 1024
EXPERT_DIM = 2048
NUM_EXPERTS = 8
K = 2
ENT = T * K
BT = 128
G_MAX = ENT // BT + NUM_EXPERTS
N_PAD = G_MAX * BT
NW = 32
EPW = ENT // NW
NCH = EPW // 16
NCH_ALL = ENT // 16


def _gating_body(x_ref, gw_ref, gb_ref, eid_ref, wts_ref):
    x = x_ref[...]
    logits = jnp.dot(x, gw_ref[...], preferred_element_type=jnp.float32)
    logits = logits + gb_ref[...]
    iota = lax.broadcasted_iota(jnp.int32, (T, NUM_EXPERTS), 1)
    m1 = jnp.max(logits, axis=-1, keepdims=True)
    idx1 = jnp.min(jnp.where(logits == m1, iota, NUM_EXPERTS), axis=-1,
                   keepdims=True)
    masked = jnp.where(iota == idx1, -jnp.inf, logits)
    m2 = jnp.max(masked, axis=-1, keepdims=True)
    idx2 = jnp.min(jnp.where(masked == m2, iota, NUM_EXPERTS), axis=-1,
                   keepdims=True)
    e2 = jnp.exp(m2 - m1)
    s = 1.0 + e2
    eid_ref[...] = jnp.concatenate([idx1, idx2], axis=1)
    wts_ref[...] = jnp.concatenate([1.0 / s, e2 / s], axis=1)


def _gating(x_flat, gate_w, gb):
    return pl.pallas_call(
        _gating_body,
        grid=(1,),
        in_specs=[
            pl.BlockSpec((T, D_MODEL), lambda i: (0, 0)),
            pl.BlockSpec((D_MODEL, NUM_EXPERTS), lambda i: (0, 0)),
            pl.BlockSpec((1, NUM_EXPERTS), lambda i: (0, 0)),
        ],
        out_specs=[
            pl.BlockSpec((T, K), lambda i: (0, 0)),
            pl.BlockSpec((T, K), lambda i: (0, 0)),
        ],
        out_shape=[
            jax.ShapeDtypeStruct((T, K), jnp.int32),
            jax.ShapeDtypeStruct((T, K), jnp.float32),
        ],
    )(x_flat, gate_w, gb)


def _route_body(eid_hbm, tok_hbm, x_hbm, xs_hbm, dest_hbm, meta_hbm,
                eid_all_v, rank_all_v, tok_v, dest_v, base_v, rows_v, sem):
    wid = lax.axis_index("s") * 2 + lax.axis_index("c")
    pltpu.sync_copy(eid_hbm, eid_all_v)
    pltpu.sync_copy(tok_hbm.at[wid], tok_v)

    ones16 = jnp.ones((16,), jnp.int32)

    def scan_body(c, carries):
        eid16 = eid_all_v[c]
        rank16 = jnp.zeros((16,), jnp.int32)
        new = []
        for e in range(NUM_EXPERTS):
            m = eid16 == jnp.full((16,), e, jnp.int32)
            mi = jnp.where(m, ones16, ones16 - ones16)
            pc = plsc.cumsum(mi)
            ce = jnp.full((16,), carries[e], jnp.int32)
            rank16 = jnp.where(m, ce + pc - ones16, rank16)
            new.append(carries[e] + jnp.sum(mi))
        rank_all_v[c] = rank16
        return tuple(new)

    cnt = lax.fori_loop(0, NCH_ALL, scan_body,
                        tuple(jnp.int32(0) for _ in range(NUM_EXPERTS)))

    iota16 = lax.iota(jnp.int32, 16)
    cnt_v = jnp.zeros((16,), jnp.int32)
    for e in range(NUM_EXPERTS):
        cnt_v = jnp.where(iota16 == jnp.full((16,), e, jnp.int32),
                          jnp.full((16,), cnt[e], jnp.int32), cnt_v)
    p_v = ((cnt_v + (BT - 1)) // BT) * BT
    cum_v = plsc.cumsum(p_v)
    starts_v = cum_v - p_v
    base_v[...] = starts_v

    @pl.when(wid == 0)
    def _meta():
        dest_v[0] = cum_v // BT
        pltpu.sync_copy(dest_v.at[0], meta_hbm)

    for cc in range(NCH):
        cg = wid * NCH + cc
        eid16 = eid_all_v[cg]
        rank16 = rank_all_v[cg]
        dest16 = plsc.load_gather(base_v, [eid16]) + rank16
        dest_v[cc] = dest16

    pltpu.sync_copy(dest_v, dest_hbm.at[wid])

    for cc in range(NCH):
        pltpu.async_copy(x_hbm.at[tok_v.at[cc]], rows_v, sem).wait()
        pltpu.async_copy(rows_v, xs_hbm.at[dest_v.at[cc]], sem).wait()


def _route(eid3, tok3, x_flat):
    mesh = plsc.VectorSubcoreMesh(core_axis_name="c", subcore_axis_name="s")
    f = pl.kernel(
        _route_body,
        out_type=[
            jax.ShapeDtypeStruct((N_PAD, D_MODEL), jnp.float32),
            jax.ShapeDtypeStruct((NW, NCH, 16), jnp.int32),
            jax.ShapeDtypeStruct((16,), jnp.int32),
        ],
        mesh=mesh,
        scratch_types=[
            pltpu.VMEM((NCH_ALL, 16), jnp.int32),
            pltpu.VMEM((NCH_ALL, 16), jnp.int32),
            pltpu.VMEM((NCH, 16), jnp.int32),
            pltpu.VMEM((NCH, 16), jnp.int32),
            pltpu.VMEM((16,), jnp.int32),
            pltpu.VMEM((16, D_MODEL), jnp.float32),
            pltpu.SemaphoreType.DMA,
        ],
        compiler_params=pltpu.CompilerParams(needs_layout_passes=False),
    )
    return f(eid3, tok3, x_flat)


def _ffn_body(meta_ref, xs_ref, w1_ref, b1_ref, w2_ref, b2_ref, out_ref):
    g = pl.program_id(0)

    @pl.when(g < meta_ref[8])
    def _compute():
        x = xs_ref[...]
        h = jnp.dot(x, w1_ref[0], preferred_element_type=jnp.float32)
        h = jnp.maximum(h + b1_ref[0], 0.0)
        o = jnp.dot(h, w2_ref[0], preferred_element_type=jnp.float32)
        out_ref[...] = o + b2_ref[0]


def _expert_of(g, meta_ref):
    e = jnp.int32(0)
    for i in range(NUM_EXPERTS):
        e = e + (g >= meta_ref[i]).astype(jnp.int32)
    return jnp.minimum(e, NUM_EXPERTS - 1)


def _ffn(meta, xs, W1, b1r, W2, b2r):
    grid_spec = pltpu.PrefetchScalarGridSpec(
        num_scalar_prefetch=1,
        grid=(G_MAX,),
        in_specs=[
            pl.BlockSpec((BT, D_MODEL), lambda g, m: (g, 0)),
            pl.BlockSpec((1, D_MODEL, EXPERT_DIM),
                         lambda g, m: (_expert_of(g, m), 0, 0)),
            pl.BlockSpec((1, 1, EXPERT_DIM),
                         lambda g, m: (_expert_of(g, m), 0, 0)),
            pl.BlockSpec((1, EXPERT_DIM, D_MODEL),
                         lambda g, m: (_expert_of(g, m), 0, 0)),
            pl.BlockSpec((1, 1, D_MODEL),
                         lambda g, m: (_expert_of(g, m), 0, 0)),
        ],
        out_specs=pl.BlockSpec((BT, D_MODEL), lambda g, m: (g, 0)),
    )
    return pl.pallas_call(
        _ffn_body,
        grid_spec=grid_spec,
        out_shape=jax.ShapeDtypeStruct((N_PAD, D_MODEL), jnp.float32),
    )(meta, xs, W1, b1r, W2, b2r)


def _cgather_body(ys_hbm, dest_hbm, yi_hbm, dest_v, rows_v, sem):
    wid = lax.axis_index("s") * 2 + lax.axis_index("c")
    pltpu.sync_copy(dest_hbm.at[wid], dest_v)
    for cc in range(NCH):
        pltpu.async_copy(ys_hbm.at[dest_v.at[cc]], rows_v, sem).wait()
        base = (wid * NCH + cc) * 16
        pltpu.sync_copy(rows_v, yi_hbm.at[pl.ds(base, 16)])


def _cgather(ys, dest3):
    mesh = plsc.VectorSubcoreMesh(core_axis_name="c", subcore_axis_name="s")
    f = pl.kernel(
        _cgather_body,
        out_type=jax.ShapeDtypeStruct((ENT, D_MODEL), jnp.float32),
        mesh=mesh,
        scratch_types=[
            pltpu.VMEM((NCH, 16), jnp.int32),
            pltpu.VMEM((16, D_MODEL), jnp.float32),
            pltpu.SemaphoreType.DMA,
        ],
        compiler_params=pltpu.CompilerParams(needs_layout_passes=False),
    )
    return f(ys, dest3)


def _combine_body(yi_ref, w_ref, out_ref):
    w = w_ref[...]
    out_ref[...] = (w[:, :1] * yi_ref[:, :D_MODEL]
                    + w[:, 1:] * yi_ref[:, D_MODEL:])


def _combine(yi2, wts):
    BTD = 512
    return pl.pallas_call(
        _combine_body,
        grid=(T // BTD,),
        in_specs=[
            pl.BlockSpec((BTD, K * D_MODEL), lambda t: (t, 0)),
            pl.BlockSpec((BTD, K), lambda t: (t, 0)),
        ],
        out_specs=pl.BlockSpec((BTD, D_MODEL), lambda t: (t, 0)),
        out_shape=jax.ShapeDtypeStruct((T, D_MODEL), jnp.float32),
    )(yi2, wts)


def kernel(x, W1, b1, W2, b2, gate_w, gate_b, bias):
    seq_len, batch_size, d_model = x.shape
    x_flat = x.reshape(-1, d_model)
    gb = (gate_b + bias).reshape(1, NUM_EXPERTS)

    eid, wts = _gating(x_flat, gate_w, gb)
    eid3 = eid.reshape(NCH_ALL, 16)
    tok3 = (jnp.arange(ENT, dtype=jnp.int32) // K).reshape(NW, NCH, 16)

    xs, dest3, meta = _route(eid3, tok3, x_flat)
    ys = _ffn(meta, xs, W1, b1.reshape(NUM_EXPERTS, 1, EXPERT_DIM),
              W2, b2.reshape(NUM_EXPERTS, 1, D_MODEL))
    yi = _cgather(ys, dest3)
    out = _combine(yi.reshape(T, K * D_MODEL), wts)
    return out.reshape(seq_len, batch_size, d_model)


# speedup vs baseline: 1.1851x; 1.0256x over previous
import functools

import jax
import jax.numpy as jnp
from jax import lax
from jax.experimental import pallas as pl
from jax.experimental.pallas import tpu as pltpu
from jax.experimental.pallas import tpu_sc as plsc

T = 2048
D_MODEL = 1024
EXPERT_DIM = 2048
NUM_EXPERTS = 8
K = 2
ENT = T * K
BT = 256
G_MAX = ENT // BT + NUM_EXPERTS
N_PAD = G_MAX * BT
NW = 32
EPW = ENT // NW
NCH = EPW // 16
NCH_ALL = ENT // 16


def _gating_body(x_ref, gw_ref, gb_ref, eid_ref, wts_ref):
    x = x_ref[...]
    logits = jnp.dot(x, gw_ref[...], preferred_element_type=jnp.float32)
    logits = logits + gb_ref[...]
    iota = lax.broadcasted_iota(jnp.int32, (T, NUM_EXPERTS), 1)
    m1 = jnp.max(logits, axis=-1, keepdims=True)
    idx1 = jnp.min(jnp.where(logits == m1, iota, NUM_EXPERTS), axis=-1,
                   keepdims=True)
    masked = jnp.where(iota == idx1, -jnp.inf, logits)
    m2 = jnp.max(masked, axis=-1, keepdims=True)
    idx2 = jnp.min(jnp.where(masked == m2, iota, NUM_EXPERTS), axis=-1,
                   keepdims=True)
    e2 = jnp.exp(m2 - m1)
    s = 1.0 + e2
    eid_ref[...] = jnp.concatenate([idx1, idx2], axis=1)
    wts_ref[...] = jnp.concatenate([1.0 / s, e2 / s], axis=1)


def _gating(x_flat, gate_w, gb):
    return pl.pallas_call(
        _gating_body,
        grid=(1,),
        in_specs=[
            pl.BlockSpec((T, D_MODEL), lambda i: (0, 0)),
            pl.BlockSpec((D_MODEL, NUM_EXPERTS), lambda i: (0, 0)),
            pl.BlockSpec((1, NUM_EXPERTS), lambda i: (0, 0)),
        ],
        out_specs=[
            pl.BlockSpec((T, K), lambda i: (0, 0)),
            pl.BlockSpec((T, K), lambda i: (0, 0)),
        ],
        out_shape=[
            jax.ShapeDtypeStruct((T, K), jnp.int32),
            jax.ShapeDtypeStruct((T, K), jnp.float32),
        ],
    )(x_flat, gate_w, gb)


def _route_body(eid_hbm, tok_hbm, x_hbm, xs_hbm, dest_hbm, meta_hbm,
                eid_all_v, rank_all_v, tok_v, dest_v, base_v, rows_v, sem):
    wid = lax.axis_index("s") * 2 + lax.axis_index("c")
    pltpu.sync_copy(eid_hbm, eid_all_v)
    pltpu.sync_copy(tok_hbm.at[wid], tok_v)

    ones16 = jnp.ones((16,), jnp.int32)

    def scan_body(c, carries):
        eid16 = eid_all_v[c]
        rank16 = jnp.zeros((16,), jnp.int32)
        new = []
        for e in range(NUM_EXPERTS):
            m = eid16 == jnp.full((16,), e, jnp.int32)
            mi = jnp.where(m, ones16, ones16 - ones16)
            pc = plsc.cumsum(mi)
            ce = jnp.full((16,), carries[e], jnp.int32)
            rank16 = jnp.where(m, ce + pc - ones16, rank16)
            new.append(carries[e] + jnp.sum(mi))
        rank_all_v[c] = rank16
        return tuple(new)

    cnt = lax.fori_loop(0, NCH_ALL, scan_body,
                        tuple(jnp.int32(0) for _ in range(NUM_EXPERTS)))

    iota16 = lax.iota(jnp.int32, 16)
    cnt_v = jnp.zeros((16,), jnp.int32)
    for e in range(NUM_EXPERTS):
        cnt_v = jnp.where(iota16 == jnp.full((16,), e, jnp.int32),
                          jnp.full((16,), cnt[e], jnp.int32), cnt_v)
    p_v = ((cnt_v + (BT - 1)) // BT) * BT
    cum_v = plsc.cumsum(p_v)
    starts_v = cum_v - p_v
    base_v[...] = starts_v

    @pl.when(wid == 0)
    def _meta():
        dest_v[0] = cum_v // BT
        pltpu.sync_copy(dest_v.at[0], meta_hbm)

    for cc in range(NCH):
        cg = wid * NCH + cc
        eid16 = eid_all_v[cg]
        rank16 = rank_all_v[cg]
        dest16 = plsc.load_gather(base_v, [eid16]) + rank16
        dest_v[cc] = dest16

    pltpu.sync_copy(dest_v, dest_hbm.at[wid])

    for cc in range(NCH):
        pltpu.async_copy(x_hbm.at[tok_v.at[cc]], rows_v, sem).wait()
        pltpu.async_copy(rows_v, xs_hbm.at[dest_v.at[cc]], sem).wait()


def _route(eid3, tok3, x_flat):
    mesh = plsc.VectorSubcoreMesh(core_axis_name="c", subcore_axis_name="s")
    f = pl.kernel(
        _route_body,
        out_type=[
            jax.ShapeDtypeStruct((N_PAD, D_MODEL), jnp.float32),
            jax.ShapeDtypeStruct((NW, NCH, 16), jnp.int32),
            jax.ShapeDtypeStruct((16,), jnp.int32),
        ],
        mesh=mesh,
        scratch_types=[
            pltpu.VMEM((NCH_ALL, 16), jnp.int32),
            pltpu.VMEM((NCH_ALL, 16), jnp.int32),
            pltpu.VMEM((NCH, 16), jnp.int32),
            pltpu.VMEM((NCH, 16), jnp.int32),
            pltpu.VMEM((16,), jnp.int32),
            pltpu.VMEM((16, D_MODEL), jnp.float32),
            pltpu.SemaphoreType.DMA,
        ],
        compiler_params=pltpu.CompilerParams(needs_layout_passes=False),
    )
    return f(eid3, tok3, x_flat)


def _ffn_body(meta_ref, xs_ref, w1_ref, b1_ref, w2_ref, b2_ref, out_ref):
    g = pl.program_id(0)

    @pl.when(g < meta_ref[8])
    def _compute():
        x = xs_ref[...]
        h = jnp.dot(x, w1_ref[0], preferred_element_type=jnp.float32)
        h = jnp.maximum(h + b1_ref[0], 0.0)
        o = jnp.dot(h, w2_ref[0], preferred_element_type=jnp.float32)
        out_ref[...] = o + b2_ref[0]


def _expert_of(g, meta_ref):
    e = jnp.int32(0)
    for i in range(NUM_EXPERTS):
        e = e + (g >= meta_ref[i]).astype(jnp.int32)
    return jnp.minimum(e, NUM_EXPERTS - 1)


def _ffn(meta, xs, W1, b1r, W2, b2r):
    grid_spec = pltpu.PrefetchScalarGridSpec(
        num_scalar_prefetch=1,
        grid=(G_MAX,),
        in_specs=[
            pl.BlockSpec((BT, D_MODEL), lambda g, m: (g, 0)),
            pl.BlockSpec((1, D_MODEL, EXPERT_DIM),
                         lambda g, m: (_expert_of(g, m), 0, 0)),
            pl.BlockSpec((1, 1, EXPERT_DIM),
                         lambda g, m: (_expert_of(g, m), 0, 0)),
            pl.BlockSpec((1, EXPERT_DIM, D_MODEL),
                         lambda g, m: (_expert_of(g, m), 0, 0)),
            pl.BlockSpec((1, 1, D_MODEL),
                         lambda g, m: (_expert_of(g, m), 0, 0)),
        ],
        out_specs=pl.BlockSpec((BT, D_MODEL), lambda g, m: (g, 0)),
    )
    return pl.pallas_call(
        _ffn_body,
        grid_spec=grid_spec,
        out_shape=jax.ShapeDtypeStruct((N_PAD, D_MODEL), jnp.float32),
        compiler_params=pltpu.CompilerParams(
            vmem_limit_bytes=120 * 1024 * 1024),
    )(meta, xs, W1, b1r, W2, b2r)


def _cgather_body(ys_hbm, dest_hbm, yi_hbm, dest_v, rows_v, sem):
    wid = lax.axis_index("s") * 2 + lax.axis_index("c")
    pltpu.sync_copy(dest_hbm.at[wid], dest_v)
    for cc in range(NCH):
        pltpu.async_copy(ys_hbm.at[dest_v.at[cc]], rows_v, sem).wait()
        base = (wid * NCH + cc) * 16
        pltpu.sync_copy(rows_v, yi_hbm.at[pl.ds(base, 16)])


def _cgather(ys, dest3):
    mesh = plsc.VectorSubcoreMesh(core_axis_name="c", subcore_axis_name="s")
    f = pl.kernel(
        _cgather_body,
        out_type=jax.ShapeDtypeStruct((ENT, D_MODEL), jnp.float32),
        mesh=mesh,
        scratch_types=[
            pltpu.VMEM((NCH, 16), jnp.int32),
            pltpu.VMEM((16, D_MODEL), jnp.float32),
            pltpu.SemaphoreType.DMA,
        ],
        compiler_params=pltpu.CompilerParams(needs_layout_passes=False),
    )
    return f(ys, dest3)


def _combine_body(yi_ref, w_ref, out_ref):
    w = w_ref[...]
    out_ref[...] = (w[:, :1] * yi_ref[:, :D_MODEL]
                    + w[:, 1:] * yi_ref[:, D_MODEL:])


def _combine(yi2, wts):
    BTD = 512
    return pl.pallas_call(
        _combine_body,
        grid=(T // BTD,),
        in_specs=[
            pl.BlockSpec((BTD, K * D_MODEL), lambda t: (t, 0)),
            pl.BlockSpec((BTD, K), lambda t: (t, 0)),
        ],
        out_specs=pl.BlockSpec((BTD, D_MODEL), lambda t: (t, 0)),
        out_shape=jax.ShapeDtypeStruct((T, D_MODEL), jnp.float32),
    )(yi2, wts)


def kernel(x, W1, b1, W2, b2, gate_w, gate_b, bias):
    seq_len, batch_size, d_model = x.shape
    x_flat = x.reshape(-1, d_model)
    gb = (gate_b + bias).reshape(1, NUM_EXPERTS)

    eid, wts = _gating(x_flat, gate_w, gb)
    eid3 = eid.reshape(NCH_ALL, 16)
    tok3 = (jnp.arange(ENT, dtype=jnp.int32) // K).reshape(NW, NCH, 16)

    xs, dest3, meta = _route(eid3, tok3, x_flat)
    ys = _ffn(meta, xs, W1, b1.reshape(NUM_EXPERTS, 1, EXPERT_DIM),
              W2, b2.reshape(NUM_EXPERTS, 1, D_MODEL))
    yi = _cgather(ys, dest3)
    out = _combine(yi.reshape(T, K * D_MODEL), wts)
    return out.reshape(seq_len, batch_size, d_model)
